# baseline (speedup 1.0000x reference)
import jax
import jax.numpy as jnp
from jax.experimental import pallas as pl


def _seg_softmax(msg, dst, n):
    m = jax.ops.segment_max(msg, dst, num_segments=n)
    m = jnp.where(jnp.isfinite(m), m, 0.0)
    ex = jnp.exp(msg - m[dst])
    denom = jax.ops.segment_sum(ex, dst, num_segments=n)
    num = jax.ops.segment_sum(ex * msg, dst, num_segments=n)
    return num / (denom + 1e-16)


def _gc(h, src, dst, W1, b1, W2, b2):
    msg = jax.nn.relu(h[src]) + 1e-7
    aggr = _seg_softmax(msg, dst, h.shape[0])
    out = h + aggr
    return jax.nn.relu(out @ W1 + b1) @ W2 + b2


def _id_kernel(x_ref, o_ref):
    o_ref[...] = x_ref[...]


def kernel(x, edge_index, Wl, bl, Win1, bin1, Win2, bin2, W0_1, b0_1, W0_2, b0_2, W1_1, b1_1, W1_2, b1_2, W2_1, b2_1, W2_2, b2_2, Wh1, bh1, Wh2, bh2):
    src = edge_index[0]
    dst = edge_index[1]
    h = x @ Wl + bl
    h = _gc(h, src, dst, Win1, bin1, Win2, bin2)
    h = jax.nn.relu(h)
    for (W1, b1, W2, b2) in ((W0_1, b0_1, W0_2, b0_2), (W1_1, b1_1, W1_2, b1_2), (W2_1, b2_1, W2_2, b2_2)):
        t = jax.nn.relu(h)
        t = _gc(t, src, dst, W1, b1, W2, b2)
        h = h + t
    pooled = jnp.max(h, axis=0, keepdims=True)
    out = jax.nn.relu(pooled @ Wh1 + bh1) @ Wh2 + bh2
    out = pl.pallas_call(
        _id_kernel,
        out_shape=jax.ShapeDtypeStruct(out.shape, out.dtype),
    )(out)
    return out


# speedup vs baseline: 5.3573x; 5.3573x over previous
import functools

import jax
import jax.numpy as jnp
from jax import lax
from jax.experimental import pallas as pl
from jax.experimental.pallas import tpu as pltpu
from jax.experimental.pallas import tpu_sc as plsc

N = 50000
HID = 64
FF = 128
OUT_DIM = 80

NC = 2
NS = 16
L = 16

BN = 400
GRID = N // BN

CHUNK = 128
GI = 8
EPAD_UNIT = NS * CHUNK * GI
E_TOTAL = 800000
EPAD = ((E_TOTAL + EPAD_UNIT - 1) // EPAD_UNIT) * EPAD_UNIT
CROWS = EPAD // CHUNK
CROWS_TILE = CROWS // NS
GROUPS = CROWS_TILE // GI

WB_ROWS = 392
WB_STEPS = 8
ACC_TILE = WB_ROWS * WB_STEPS
NACC = NS * ACC_TILE

_HIGH = jax.lax.Precision.HIGHEST


def _dot(a, b):
    return jnp.dot(a, b, preferred_element_type=jnp.float32, precision=_HIGH)


def _k0_body(x_ref, w_ref, b_ref, h_ref, m_ref):
    j = pl.program_id(0)
    h = _dot(x_ref[...], w_ref[...]) + b_ref[0:1, :]
    h_ref[...] = h
    v = jax.nn.relu(h) + 1e-7
    bm = jnp.broadcast_to(jnp.max(v, axis=0, keepdims=True), (8, HID))

    @pl.when(j == 0)
    def _():
        m_ref[...] = bm

    @pl.when(j > 0)
    def _():
        m_ref[...] = jnp.maximum(m_ref[...], bm)


def _run_k0(xp, Wlp, bl2):
    return pl.pallas_call(
        _k0_body,
        grid=(GRID,),
        in_specs=[
            pl.BlockSpec((BN, 8), lambda j: (j, 0)),
            pl.BlockSpec((8, HID), lambda j: (0, 0)),
            pl.BlockSpec((8, HID), lambda j: (0, 0)),
        ],
        out_specs=[
            pl.BlockSpec((BN, HID), lambda j: (j, 0)),
            pl.BlockSpec((8, HID), lambda j: (0, 0)),
        ],
        out_shape=[
            jax.ShapeDtypeStruct((N, HID), jnp.float32),
            jax.ShapeDtypeStruct((8, HID), jnp.float32),
        ],
    )(xp, Wlp, bl2)


def _tbl_body(h_ref, m_ref, *out_refs):
    v = jax.nn.relu(h_ref[...]) + 1e-7
    w = jnp.exp(v - m_ref[0:1, :])
    wv = w * v
    for cb in range(4):
        out_refs[cb][...] = w[:, cb * L:(cb + 1) * L]
        out_refs[4 + cb][...] = wv[:, cb * L:(cb + 1) * L]


def _run_tbl(h, M):
    return pl.pallas_call(
        _tbl_body,
        grid=(GRID,),
        in_specs=[
            pl.BlockSpec((BN, HID), lambda j: (j, 0)),
            pl.BlockSpec((8, HID), lambda j: (0, 0)),
        ],
        out_specs=[pl.BlockSpec((BN, L), lambda j: (j, 0))] * 8,
        out_shape=[jax.ShapeDtypeStruct((N, L), jnp.float32)] * 8,
    )(h, M)


def _sc_body(src_ref, dst_ref,
             td0, td1, td2, td3, tn0, tn1, tn2, tn3,
             den0, den1, den2, den3, num0, num1, num2, num3,
             accA, accB, zbuf, sidx, didx, gd, gn, wbuf, semd, semn):
    c = lax.axis_index("c")
    s = lax.axis_index("s")
    tds = (td0, td1, td2, td3)
    tns = (tn0, tn1, tn2, tn3)
    dens = (den0, den1, den2, den3)
    nums = (num0, num1, num2, num3)
    row0 = s * ACC_TILE
    chunk0 = s * CROWS_TILE

    def _zb(i, _):
        zbuf[i, :] = jnp.zeros((L,), jnp.float32)
        return _
    lax.fori_loop(0, WB_ROWS, _zb, None)

    def _edges(td, tn):
        def grp(g, _):
            r = chunk0 + g * GI
            pltpu.sync_copy(src_ref.at[pl.ds(r, GI)], sidx)
            pltpu.sync_copy(dst_ref.at[pl.ds(r, GI)], didx)
            for j in range(GI):
                cp1 = pltpu.async_copy(td.at[sidx.at[j]], gd, semd)
                cp2 = pltpu.async_copy(tn.at[sidx.at[j]], gn, semn)
                cp1.wait()
                cp2.wait()
                pltpu.sync_copy(gd, accA.at[didx.at[j]], add=True)
                pltpu.sync_copy(gn, accB.at[didx.at[j]], add=True)
            return _
        lax.fori_loop(0, GROUPS, grp, None)

    def _writeback(den, num):
        def wr(k, _):
            base = row0 + k * WB_ROWS
            pltpu.sync_copy(accA.at[pl.ds(base, WB_ROWS)], wbuf)
            pltpu.sync_copy(wbuf, den.at[pl.ds(base, WB_ROWS)])
            pltpu.sync_copy(accB.at[pl.ds(base, WB_ROWS)], wbuf)
            pltpu.sync_copy(wbuf, num.at[pl.ds(base, WB_ROWS)])
            return _
        lax.fori_loop(0, WB_STEPS, wr, None)

    for phase in range(2):
        def _zr(k, _):
            base = row0 + k * WB_ROWS
            pltpu.sync_copy(zbuf, accA.at[pl.ds(base, WB_ROWS)])
            pltpu.sync_copy(zbuf, accB.at[pl.ds(base, WB_ROWS)])
            return _
        lax.fori_loop(0, WB_STEPS, _zr, None)
        plsc.subcore_barrier()

        for cc in range(NC):
            cb = 2 * cc + phase

            @pl.when(c == cc)
            def _(cb=cb):
                _edges(tds[cb], tns[cb])
        plsc.subcore_barrier()

        for cc in range(NC):
            cb = 2 * cc + phase

            @pl.when(c == cc)
            def _(cb=cb):
                _writeback(dens[cb], nums[cb])
        plsc.subcore_barrier()


def _run_sc(src2d, dst2d, tabs):
    f = pl.kernel(
        _sc_body,
        out_type=[jax.ShapeDtypeStruct((NACC, L), jnp.float32)] * 8,
        mesh=plsc.VectorSubcoreMesh(core_axis_name="c", subcore_axis_name="s"),
        compiler_params=pltpu.CompilerParams(use_tc_tiling_on_sc=False),
        scratch_types=[
            pltpu.VMEM_SHARED((NACC, L), jnp.float32),
            pltpu.VMEM_SHARED((NACC, L), jnp.float32),
            pltpu.VMEM((WB_ROWS, L), jnp.float32),
            pltpu.VMEM((GI, CHUNK), jnp.int32),
            pltpu.VMEM((GI, CHUNK), jnp.int32),
            pltpu.VMEM((CHUNK, L), jnp.float32),
            pltpu.VMEM((CHUNK, L), jnp.float32),
            pltpu.VMEM((WB_ROWS, L), jnp.float32),
            pltpu.SemaphoreType.DMA,
            pltpu.SemaphoreType.DMA,
        ],
    )
    return f(src2d, dst2d, *tabs)


def _layer_body(h_ref, m_ref, w1_ref, b1_ref, w2_ref, b2_ref,
                d0, d1, d2, d3, n0, n1, n2, n3,
                h_out, m_out, *, first):
    j = pl.program_id(0)
    eps = jnp.maximum(1e-16 * jnp.exp(-m_ref[0:1, :]), 1e-38)
    dd = (d0, d1, d2, d3)
    nn = (n0, n1, n2, n3)
    aggr = jnp.concatenate(
        [nn[cb][...] / (dd[cb][...] + eps[:, cb * L:(cb + 1) * L])
         for cb in range(4)], axis=1)
    h = h_ref[...]
    base = h if first else jax.nn.relu(h)
    u = base + aggr
    t = _dot(jax.nn.relu(_dot(u, w1_ref[...]) + b1_ref[0:1, :]),
             w2_ref[...]) + b2_ref[0:1, :]
    hnew = jax.nn.relu(t) if first else h + t
    h_out[...] = hnew
    v = jax.nn.relu(hnew) + 1e-7
    bm = jnp.broadcast_to(jnp.max(v, axis=0, keepdims=True), (8, HID))

    @pl.when(j == 0)
    def _():
        m_out[...] = bm

    @pl.when(j > 0)
    def _():
        m_out[...] = jnp.maximum(m_out[...], bm)


def _run_layer(h, M, W1, b1, W2, b2, dens, nums, first):
    return pl.pallas_call(
        functools.partial(_layer_body, first=first),
        grid=(GRID,),
        in_specs=[
            pl.BlockSpec((BN, HID), lambda j: (j, 0)),
            pl.BlockSpec((8, HID), lambda j: (0, 0)),
            pl.BlockSpec((HID, FF), lambda j: (0, 0)),
            pl.BlockSpec((8, FF), lambda j: (0, 0)),
            pl.BlockSpec((FF, HID), lambda j: (0, 0)),
            pl.BlockSpec((8, HID), lambda j: (0, 0)),
        ] + [pl.BlockSpec((BN, L), lambda j: (j, 0))] * 8,
        out_specs=[
            pl.BlockSpec((BN, HID), lambda j: (j, 0)),
            pl.BlockSpec((8, HID), lambda j: (0, 0)),
        ],
        out_shape=[
            jax.ShapeDtypeStruct((N, HID), jnp.float32),
            jax.ShapeDtypeStruct((8, HID), jnp.float32),
        ],
    )(h, M, W1, b1, W2, b2, *dens, *nums)


def _final_body(h_ref, m_ref, w1_ref, b1_ref, w2_ref, b2_ref,
                wh1_ref, bh1_ref, wh2_ref, bh2_ref,
                d0, d1, d2, d3, n0, n1, n2, n3,
                out_ref, pool_ref):
    j = pl.program_id(0)
    eps = jnp.maximum(1e-16 * jnp.exp(-m_ref[0:1, :]), 1e-38)
    dd = (d0, d1, d2, d3)
    nn = (n0, n1, n2, n3)
    aggr = jnp.concatenate(
        [nn[cb][...] / (dd[cb][...] + eps[:, cb * L:(cb + 1) * L])
         for cb in range(4)], axis=1)
    h = h_ref[...]
    u = jax.nn.relu(h) + aggr
    t = _dot(jax.nn.relu(_dot(u, w1_ref[...]) + b1_ref[0:1, :]),
             w2_ref[...]) + b2_ref[0:1, :]
    hnew = h + t
    bm = jnp.broadcast_to(jnp.max(hnew, axis=0, keepdims=True), (8, HID))

    @pl.when(j == 0)
    def _():
        pool_ref[...] = bm

    @pl.when(j > 0)
    def _():
        pool_ref[...] = jnp.maximum(pool_ref[...], bm)

    @pl.when(j == GRID - 1)
    def _():
        pooled = pool_ref[...]
        z = jax.nn.relu(_dot(pooled, wh1_ref[...]) + bh1_ref[0:1, :])
        out_ref[...] = _dot(z, wh2_ref[...]) + bh2_ref[0:1, :]


def _run_final(h, M, W1, b1, W2, b2, Wh1, bh1, Wh2, bh2, dens, nums):
    return pl.pallas_call(
        _final_body,
        grid=(GRID,),
        in_specs=[
            pl.BlockSpec((BN, HID), lambda j: (j, 0)),
            pl.BlockSpec((8, HID), lambda j: (0, 0)),
            pl.BlockSpec((HID, FF), lambda j: (0, 0)),
            pl.BlockSpec((8, FF), lambda j: (0, 0)),
            pl.BlockSpec((FF, HID), lambda j: (0, 0)),
            pl.BlockSpec((8, HID), lambda j: (0, 0)),
            pl.BlockSpec((HID, HID), lambda j: (0, 0)),
            pl.BlockSpec((8, HID), lambda j: (0, 0)),
            pl.BlockSpec((HID, OUT_DIM), lambda j: (0, 0)),
            pl.BlockSpec((8, OUT_DIM), lambda j: (0, 0)),
        ] + [pl.BlockSpec((BN, L), lambda j: (j, 0))] * 8,
        out_specs=[pl.BlockSpec((8, OUT_DIM), lambda j: (0, 0))],
        out_shape=[jax.ShapeDtypeStruct((8, OUT_DIM), jnp.float32)],
        scratch_shapes=[pltpu.VMEM((8, HID), jnp.float32)],
    )(h, M, W1, b1, W2, b2, Wh1, bh1, Wh2, bh2, *dens, *nums)


def _b8(b):
    return jnp.broadcast_to(b[None, :], (8, b.shape[0]))


def kernel(x, edge_index, Wl, bl, Win1, bin1, Win2, bin2,
           W0_1, b0_1, W0_2, b0_2, W1_1, b1_1, W1_2, b1_2,
           W2_1, b2_1, W2_2, b2_2, Wh1, bh1, Wh2, bh2):
    xp = jnp.pad(x, ((0, 0), (0, 2)))
    Wlp = jnp.pad(Wl, ((0, 2), (0, 0)))
    src = edge_index[0]
    dst = edge_index[1]
    pad = EPAD - src.shape[0]
    src2d = jnp.concatenate(
        [src, jnp.zeros((pad,), jnp.int32)]).reshape(CROWS, CHUNK)
    dst2d = jnp.concatenate(
        [dst, jnp.full((pad,), N, jnp.int32)]).reshape(CROWS, CHUNK)

    h, M = _run_k0(xp, Wlp, _b8(bl))

    layers = [
        (Win1, bin1, Win2, bin2),
        (W0_1, b0_1, W0_2, b0_2),
        (W1_1, b1_1, W1_2, b1_2),
        (W2_1, b2_1, W2_2, b2_2),
    ]
    for li, (W1, b1, W2, b2) in enumerate(layers):
        tabs = _run_tbl(h, M)
        dn = _run_sc(src2d, dst2d, tabs)
        dens, nums = dn[:4], dn[4:]
        if li < 3:
            h, M = _run_layer(h, M, W1, _b8(b1), W2, _b8(b2),
                              dens, nums, first=(li == 0))
        else:
            out8 = _run_final(h, M, W1, _b8(b1), W2, _b8(b2),
                              Wh1, _b8(bh1), Wh2, _b8(bh2), dens, nums)[0]
    return out8[0:1, :]


# speedup vs baseline: 8.1998x; 1.5306x over previous
import functools

import jax
import jax.numpy as jnp
from jax import lax
from jax.experimental import pallas as pl
from jax.experimental.pallas import tpu as pltpu
from jax.experimental.pallas import tpu_sc as plsc

N = 50000
HID = 64
FF = 128
OUT_DIM = 80

NC = 2
NS = 16
L = 16

BN = 400
GRID = N // BN

CHUNK = 128
EPAD_UNIT = NS * CHUNK * 8
E_TOTAL = 800000
EPAD = ((E_TOTAL + EPAD_UNIT - 1) // EPAD_UNIT) * EPAD_UNIT
CROWS = EPAD // CHUNK
CROWS_TILE = CROWS // NS
STAGES = 14
CH_Q = CROWS_TILE // STAGES
GRP = 2
NGRP = CH_Q // GRP
GROWS = GRP * CHUNK

WB_ROWS = 112
WB_STEPS = 28
ACC_TILE = WB_ROWS * WB_STEPS
NACC = NS * ACC_TILE

_HIGH = jax.lax.Precision.HIGHEST


def _dot(a, b):
    return jnp.dot(a, b, preferred_element_type=jnp.float32, precision=_HIGH)


def _k0_body(x_ref, w_ref, b_ref, h_ref, m_ref):
    j = pl.program_id(0)
    h = _dot(x_ref[...], w_ref[...]) + b_ref[0:1, :]
    h_ref[...] = h
    v = jax.nn.relu(h) + 1e-7
    bm = jnp.broadcast_to(jnp.max(v, axis=0, keepdims=True), (8, HID))

    @pl.when(j == 0)
    def _():
        m_ref[...] = bm

    @pl.when(j > 0)
    def _():
        m_ref[...] = jnp.maximum(m_ref[...], bm)


def _run_k0(xp, Wlp, bl2):
    return pl.pallas_call(
        _k0_body,
        grid=(GRID,),
        in_specs=[
            pl.BlockSpec((BN, 8), lambda j: (j, 0)),
            pl.BlockSpec((8, HID), lambda j: (0, 0)),
            pl.BlockSpec((8, HID), lambda j: (0, 0)),
        ],
        out_specs=[
            pl.BlockSpec((BN, HID), lambda j: (j, 0)),
            pl.BlockSpec((8, HID), lambda j: (0, 0)),
        ],
        out_shape=[
            jax.ShapeDtypeStruct((N, HID), jnp.float32),
            jax.ShapeDtypeStruct((8, HID), jnp.float32),
        ],
    )(xp, Wlp, bl2)


def _tbl_body(h_ref, m_ref, *out_refs):
    v = jax.nn.relu(h_ref[...]) + 1e-7
    w = jnp.exp(v - m_ref[0:1, :])
    wv = w * v
    for cb in range(4):
        out_refs[cb][...] = w[:, cb * L:(cb + 1) * L]
        out_refs[4 + cb][...] = wv[:, cb * L:(cb + 1) * L]


def _run_tbl(h, M):
    return pl.pallas_call(
        _tbl_body,
        grid=(GRID,),
        in_specs=[
            pl.BlockSpec((BN, HID), lambda j: (j, 0)),
            pl.BlockSpec((8, HID), lambda j: (0, 0)),
        ],
        out_specs=[pl.BlockSpec((BN, L), lambda j: (j, 0))] * 8,
        out_shape=[jax.ShapeDtypeStruct((N, L), jnp.float32)] * 8,
    )(h, M)


def _sc_body(src_ref, dst_ref,
             td0, td1, td2, td3, tn0, tn1, tn2, tn3,
             den0, den1, den2, den3, num0, num1, num2, num3,
             accA, accB, zbuf, sidx, didx, gd, gn, wbuf,
             semd, semn, semsd, semsn):
    c = lax.axis_index("c")
    s = lax.axis_index("s")
    tds = (td0, td1, td2, td3)
    tns = (tn0, tn1, tn2, tn3)
    dens = (den0, den1, den2, den3)
    nums = (num0, num1, num2, num3)
    row0 = s * ACC_TILE
    chunk0 = s * CROWS_TILE

    def _zb(i, _):
        zbuf[i, :] = jnp.zeros((L,), jnp.float32)
        return _
    lax.fori_loop(0, WB_ROWS, _zb, None)

    def _fire_gathers(td, tn, g, slot):
        off = slot * GROWS
        for j in range(GRP):
            r = g * GRP + j
            pltpu.async_copy(td.at[sidx.at[r]], gd.at[pl.ds(off + j * CHUNK, CHUNK)], semd)
            pltpu.async_copy(tn.at[sidx.at[r]], gn.at[pl.ds(off + j * CHUNK, CHUNK)], semn)

    def _fire_scatters(g, slot):
        off = slot * GROWS
        for j in range(GRP):
            r = g * GRP + j
            pltpu.async_copy(gd.at[pl.ds(off + j * CHUNK, CHUNK)],
                             accA.at[didx.at[r]], semsd, add=True)
            pltpu.async_copy(gn.at[pl.ds(off + j * CHUNK, CHUNK)],
                             accB.at[didx.at[r]], semsn, add=True)

    def _drain_gathers(td):
        pltpu.make_async_copy(td.at[pl.ds(0, GROWS)], gd.at[pl.ds(0, GROWS)], semd).wait()
        pltpu.make_async_copy(td.at[pl.ds(0, GROWS)], gn.at[pl.ds(0, GROWS)], semn).wait()

    def _drain_scatters(td):
        pltpu.make_async_copy(td.at[pl.ds(0, GROWS)], gd.at[pl.ds(0, GROWS)], semsd).wait()
        pltpu.make_async_copy(td.at[pl.ds(0, GROWS)], gn.at[pl.ds(0, GROWS)], semsn).wait()

    def _edges(td, tn):
        def stage(q, _):
            r0 = chunk0 + q * CH_Q
            pltpu.sync_copy(src_ref.at[pl.ds(r0, CH_Q)], sidx)
            pltpu.sync_copy(dst_ref.at[pl.ds(r0, CH_Q)], didx)
            _fire_gathers(td, tn, 0, 0)

            def grp(g, _):
                slot = lax.rem(g, 2)

                @pl.when(g > 0)
                def _():
                    _drain_scatters(td)

                @pl.when(g < NGRP - 1)
                def _():
                    _fire_gathers(td, tn, g + 1, 1 - slot)
                _drain_gathers(td)
                _fire_scatters(g, slot)
                return _
            lax.fori_loop(0, NGRP, grp, None)
            _drain_scatters(td)
            return _
        lax.fori_loop(0, STAGES, stage, None)

    def _writeback(den, num):
        def wr(k, _):
            base = row0 + k * WB_ROWS
            pltpu.sync_copy(accA.at[pl.ds(base, WB_ROWS)], wbuf)
            pltpu.sync_copy(wbuf, den.at[pl.ds(base, WB_ROWS)])
            pltpu.sync_copy(accB.at[pl.ds(base, WB_ROWS)], wbuf)
            pltpu.sync_copy(wbuf, num.at[pl.ds(base, WB_ROWS)])
            return _
        lax.fori_loop(0, WB_STEPS, wr, None)

    for phase in range(2):
        def _zr(k, _):
            base = row0 + k * WB_ROWS
            pltpu.sync_copy(zbuf, accA.at[pl.ds(base, WB_ROWS)])
            pltpu.sync_copy(zbuf, accB.at[pl.ds(base, WB_ROWS)])
            return _
        lax.fori_loop(0, WB_STEPS, _zr, None)
        plsc.subcore_barrier()

        for cc in range(NC):
            cb = 2 * cc + phase

            @pl.when(c == cc)
            def _(cb=cb):
                _edges(tds[cb], tns[cb])
        plsc.subcore_barrier()

        for cc in range(NC):
            cb = 2 * cc + phase

            @pl.when(c == cc)
            def _(cb=cb):
                _writeback(dens[cb], nums[cb])
        plsc.subcore_barrier()


def _run_sc(src2d, dst2d, tabs):
    f = pl.kernel(
        _sc_body,
        out_type=[jax.ShapeDtypeStruct((NACC, L), jnp.float32)] * 8,
        mesh=plsc.VectorSubcoreMesh(core_axis_name="c", subcore_axis_name="s"),
        compiler_params=pltpu.CompilerParams(use_tc_tiling_on_sc=False),
        scratch_types=[
            pltpu.VMEM_SHARED((NACC, L), jnp.float32),
            pltpu.VMEM_SHARED((NACC, L), jnp.float32),
            pltpu.VMEM((WB_ROWS, L), jnp.float32),
            pltpu.VMEM((CH_Q, CHUNK), jnp.int32),
            pltpu.VMEM((CH_Q, CHUNK), jnp.int32),
            pltpu.VMEM((2 * GROWS, L), jnp.float32),
            pltpu.VMEM((2 * GROWS, L), jnp.float32),
            pltpu.VMEM((WB_ROWS, L), jnp.float32),
            pltpu.SemaphoreType.DMA,
            pltpu.SemaphoreType.DMA,
            pltpu.SemaphoreType.DMA,
            pltpu.SemaphoreType.DMA,
        ],
    )
    return f(src2d, dst2d, *tabs)


def _layer_body(h_ref, m_ref, w1_ref, b1_ref, w2_ref, b2_ref,
                d0, d1, d2, d3, n0, n1, n2, n3,
                h_out, m_out, *, first):
    j = pl.program_id(0)
    eps = jnp.maximum(1e-16 * jnp.exp(-m_ref[0:1, :]), 1e-38)
    dd = (d0, d1, d2, d3)
    nn = (n0, n1, n2, n3)
    aggr = jnp.concatenate(
        [nn[cb][...] / (dd[cb][...] + eps[:, cb * L:(cb + 1) * L])
         for cb in range(4)], axis=1)
    h = h_ref[...]
    base = h if first else jax.nn.relu(h)
    u = base + aggr
    t = _dot(jax.nn.relu(_dot(u, w1_ref[...]) + b1_ref[0:1, :]),
             w2_ref[...]) + b2_ref[0:1, :]
    hnew = jax.nn.relu(t) if first else h + t
    h_out[...] = hnew
    v = jax.nn.relu(hnew) + 1e-7
    bm = jnp.broadcast_to(jnp.max(v, axis=0, keepdims=True), (8, HID))

    @pl.when(j == 0)
    def _():
        m_out[...] = bm

    @pl.when(j > 0)
    def _():
        m_out[...] = jnp.maximum(m_out[...], bm)


def _run_layer(h, M, W1, b1, W2, b2, dens, nums, first):
    return pl.pallas_call(
        functools.partial(_layer_body, first=first),
        grid=(GRID,),
        in_specs=[
            pl.BlockSpec((BN, HID), lambda j: (j, 0)),
            pl.BlockSpec((8, HID), lambda j: (0, 0)),
            pl.BlockSpec((HID, FF), lambda j: (0, 0)),
            pl.BlockSpec((8, FF), lambda j: (0, 0)),
            pl.BlockSpec((FF, HID), lambda j: (0, 0)),
            pl.BlockSpec((8, HID), lambda j: (0, 0)),
        ] + [pl.BlockSpec((BN, L), lambda j: (j, 0))] * 8,
        out_specs=[
            pl.BlockSpec((BN, HID), lambda j: (j, 0)),
            pl.BlockSpec((8, HID), lambda j: (0, 0)),
        ],
        out_shape=[
            jax.ShapeDtypeStruct((N, HID), jnp.float32),
            jax.ShapeDtypeStruct((8, HID), jnp.float32),
        ],
    )(h, M, W1, b1, W2, b2, *dens, *nums)


def _final_body(h_ref, m_ref, w1_ref, b1_ref, w2_ref, b2_ref,
                wh1_ref, bh1_ref, wh2_ref, bh2_ref,
                d0, d1, d2, d3, n0, n1, n2, n3,
                out_ref, pool_ref):
    j = pl.program_id(0)
    eps = jnp.maximum(1e-16 * jnp.exp(-m_ref[0:1, :]), 1e-38)
    dd = (d0, d1, d2, d3)
    nn = (n0, n1, n2, n3)
    aggr = jnp.concatenate(
        [nn[cb][...] / (dd[cb][...] + eps[:, cb * L:(cb + 1) * L])
         for cb in range(4)], axis=1)
    h = h_ref[...]
    u = jax.nn.relu(h) + aggr
    t = _dot(jax.nn.relu(_dot(u, w1_ref[...]) + b1_ref[0:1, :]),
             w2_ref[...]) + b2_ref[0:1, :]
    hnew = h + t
    bm = jnp.broadcast_to(jnp.max(hnew, axis=0, keepdims=True), (8, HID))

    @pl.when(j == 0)
    def _():
        pool_ref[...] = bm

    @pl.when(j > 0)
    def _():
        pool_ref[...] = jnp.maximum(pool_ref[...], bm)

    @pl.when(j == GRID - 1)
    def _():
        pooled = pool_ref[...]
        z = jax.nn.relu(_dot(pooled, wh1_ref[...]) + bh1_ref[0:1, :])
        out_ref[...] = _dot(z, wh2_ref[...]) + bh2_ref[0:1, :]


def _run_final(h, M, W1, b1, W2, b2, Wh1, bh1, Wh2, bh2, dens, nums):
    return pl.pallas_call(
        _final_body,
        grid=(GRID,),
        in_specs=[
            pl.BlockSpec((BN, HID), lambda j: (j, 0)),
            pl.BlockSpec((8, HID), lambda j: (0, 0)),
            pl.BlockSpec((HID, FF), lambda j: (0, 0)),
            pl.BlockSpec((8, FF), lambda j: (0, 0)),
            pl.BlockSpec((FF, HID), lambda j: (0, 0)),
            pl.BlockSpec((8, HID), lambda j: (0, 0)),
            pl.BlockSpec((HID, HID), lambda j: (0, 0)),
            pl.BlockSpec((8, HID), lambda j: (0, 0)),
            pl.BlockSpec((HID, OUT_DIM), lambda j: (0, 0)),
            pl.BlockSpec((8, OUT_DIM), lambda j: (0, 0)),
        ] + [pl.BlockSpec((BN, L), lambda j: (j, 0))] * 8,
        out_specs=[pl.BlockSpec((8, OUT_DIM), lambda j: (0, 0))],
        out_shape=[jax.ShapeDtypeStruct((8, OUT_DIM), jnp.float32)],
        scratch_shapes=[pltpu.VMEM((8, HID), jnp.float32)],
    )(h, M, W1, b1, W2, b2, Wh1, bh1, Wh2, bh2, *dens, *nums)


def _b8(b):
    return jnp.broadcast_to(b[None, :], (8, b.shape[0]))


def kernel(x, edge_index, Wl, bl, Win1, bin1, Win2, bin2,
           W0_1, b0_1, W0_2, b0_2, W1_1, b1_1, W1_2, b1_2,
           W2_1, b2_1, W2_2, b2_2, Wh1, bh1, Wh2, bh2):
    xp = jnp.pad(x, ((0, 0), (0, 2)))
    Wlp = jnp.pad(Wl, ((0, 2), (0, 0)))
    src = edge_index[0]
    dst = edge_index[1]
    pad = EPAD - src.shape[0]
    src2d = jnp.concatenate(
        [src, jnp.zeros((pad,), jnp.int32)]).reshape(CROWS, CHUNK)
    dst2d = jnp.concatenate(
        [dst, jnp.full((pad,), N, jnp.int32)]).reshape(CROWS, CHUNK)

    h, M = _run_k0(xp, Wlp, _b8(bl))

    layers = [
        (Win1, bin1, Win2, bin2),
        (W0_1, b0_1, W0_2, b0_2),
        (W1_1, b1_1, W1_2, b1_2),
        (W2_1, b2_1, W2_2, b2_2),
    ]
    for li, (W1, b1, W2, b2) in enumerate(layers):
        tabs = _run_tbl(h, M)
        dn = _run_sc(src2d, dst2d, tabs)
        dens, nums = dn[:4], dn[4:]
        if li < 3:
            h, M = _run_layer(h, M, W1, _b8(b1), W2, _b8(b2),
                              dens, nums, first=(li == 0))
        else:
            out8 = _run_final(h, M, W1, _b8(b1), W2, _b8(b2),
                              Wh1, _b8(bh1), Wh2, _b8(bh2), dens, nums)[0]
    return out8[0:1, :]


# speedup vs baseline: 13.8091x; 1.6841x over previous
import functools

import jax
import jax.numpy as jnp
from jax import lax
from jax.experimental import pallas as pl
from jax.experimental.pallas import tpu as pltpu
from jax.experimental.pallas import tpu_sc as plsc

N = 50000
HID = 64
FF = 128
OUT_DIM = 80

NC = 2
NS = 16
L = 16

BN = 2000
GRID = N // BN

CHUNK = 128
EPAD_UNIT = NS * CHUNK * 8
E_TOTAL = 800000
EPAD = ((E_TOTAL + EPAD_UNIT - 1) // EPAD_UNIT) * EPAD_UNIT
CROWS = EPAD // CHUNK
CROWS_TILE = CROWS // NS
STAGES = 14
CH_Q = CROWS_TILE // STAGES
GRP = 2
NGRP = CH_Q // GRP
GROWS = GRP * CHUNK

WB_ROWS = 112
WB_STEPS = 28
ACC_TILE = WB_ROWS * WB_STEPS
NACC = NS * ACC_TILE
ZB_ROWS = 56
ZB_STEPS = ACC_TILE // ZB_ROWS


def _dot(a, b):
    return jnp.dot(a, b, preferred_element_type=jnp.float32)


def _colmax8(v):
    return jnp.broadcast_to(jnp.max(v, axis=0, keepdims=True), (8, HID))


def _k0_body(x_ref, w_ref, b_ref, h_ref, m_ref):
    j = pl.program_id(0)
    h = _dot(x_ref[...], w_ref[...]) + b_ref[0:1, :]
    h_ref[...] = h
    bm = _colmax8(jax.nn.relu(h) + 1e-7)

    @pl.when(j == 0)
    def _():
        m_ref[...] = bm

    @pl.when(j > 0)
    def _():
        m_ref[...] = jnp.maximum(m_ref[...], bm)


def _run_k0(xp, Wlp, bl2):
    return pl.pallas_call(
        _k0_body,
        grid=(GRID,),
        in_specs=[
            pl.BlockSpec((BN, 8), lambda j: (j, 0)),
            pl.BlockSpec((8, HID), lambda j: (0, 0)),
            pl.BlockSpec((8, HID), lambda j: (0, 0)),
        ],
        out_specs=[
            pl.BlockSpec((BN, HID), lambda j: (j, 0)),
            pl.BlockSpec((8, HID), lambda j: (0, 0)),
        ],
        out_shape=[
            jax.ShapeDtypeStruct((N, HID), jnp.float32),
            jax.ShapeDtypeStruct((8, HID), jnp.float32),
        ],
    )(xp, Wlp, bl2)


def _tbl_body(h_ref, m_ref, t_ref):
    v = jax.nn.relu(h_ref[...]) + 1e-7
    w = jnp.exp(v - m_ref[0:1, :])
    wv = w * v
    parts = []
    for cb in range(4):
        parts.append(w[:, cb * L:(cb + 1) * L])
        parts.append(wv[:, cb * L:(cb + 1) * L])
    t_ref[...] = jnp.concatenate(parts, axis=1)


def _run_tbl(h, M):
    return pl.pallas_call(
        _tbl_body,
        grid=(GRID,),
        in_specs=[
            pl.BlockSpec((BN, HID), lambda j: (j, 0)),
            pl.BlockSpec((8, HID), lambda j: (0, 0)),
        ],
        out_specs=[pl.BlockSpec((BN, 2 * HID), lambda j: (j, 0))],
        out_shape=[jax.ShapeDtypeStruct((N, 2 * HID), jnp.float32)],
    )(h, M)[0]


def _sc_body(g0_ref, g1_ref, g2_ref, g3_ref, dst_ref, tab_ref, agg_ref,
             accC, zbuf, sidx, didx, gb, wbuf, semg, sems):
    c = lax.axis_index("c")
    s = lax.axis_index("s")
    gidx = (g0_ref, g1_ref, g2_ref, g3_ref)
    row0 = s * ACC_TILE
    chunk0 = s * CROWS_TILE

    def _zb(i, _):
        zbuf[i, 0:L] = jnp.zeros((L,), jnp.float32)
        zbuf[i, L:2 * L] = jnp.zeros((L,), jnp.float32)
        return _
    lax.fori_loop(0, ZB_ROWS, _zb, None)

    def _fire_gathers(gsrc, g, slot):
        off = slot * GROWS
        for j in range(GRP):
            r = g * GRP + j
            pltpu.async_copy(tab_ref.at[sidx.at[r]],
                             gb.at[pl.ds(off + j * CHUNK, CHUNK)], semg)

    def _fire_scatters(g, slot):
        off = slot * GROWS
        for j in range(GRP):
            r = g * GRP + j
            pltpu.async_copy(gb.at[pl.ds(off + j * CHUNK, CHUNK)],
                             accC.at[didx.at[r]], sems, add=True)

    def _drain(sem):
        pltpu.make_async_copy(tab_ref.at[pl.ds(0, GROWS)],
                              gb.at[pl.ds(0, GROWS)], sem).wait()

    def _edges(gsrc):
        def stage(q, _):
            r0 = chunk0 + q * CH_Q
            pltpu.sync_copy(gsrc.at[pl.ds(r0, CH_Q)], sidx)
            pltpu.sync_copy(dst_ref.at[pl.ds(r0, CH_Q)], didx)
            _fire_gathers(gsrc, 0, 0)

            def grp(g, _):
                slot = lax.rem(g, 2)

                @pl.when(g > 0)
                def _():
                    _drain(sems)

                @pl.when(g < NGRP - 1)
                def _():
                    _fire_gathers(gsrc, g + 1, 1 - slot)
                _drain(semg)
                _fire_scatters(g, slot)
                return _
            lax.fori_loop(0, NGRP, grp, None)
            _drain(sems)
            return _
        lax.fori_loop(0, STAGES, stage, None)

    def _writeback(cb):
        def wr(k, _):
            base = row0 + k * WB_ROWS
            pltpu.sync_copy(accC.at[pl.ds(base, WB_ROWS)], wbuf)
            pltpu.sync_copy(
                wbuf, agg_ref.at[pl.ds(base, WB_ROWS), pl.ds(cb * 2 * L, 2 * L)])
            return _
        lax.fori_loop(0, WB_STEPS, wr, None)

    for phase in range(2):
        def _zr(k, _):
            base = row0 + k * ZB_ROWS
            pltpu.sync_copy(zbuf, accC.at[pl.ds(base, ZB_ROWS)])
            return _
        lax.fori_loop(0, ZB_STEPS, _zr, None)
        plsc.subcore_barrier()

        for cc in range(NC):
            cb = 2 * cc + phase

            @pl.when(c == cc)
            def _(cb=cb):
                _edges(gidx[cb])
        plsc.subcore_barrier()

        for cc in range(NC):
            cb = 2 * cc + phase

            @pl.when(c == cc)
            def _(cb=cb):
                _writeback(cb)
        plsc.subcore_barrier()


def _run_sc(gidx, dst2d, tab4):
    f = pl.kernel(
        _sc_body,
        out_type=[jax.ShapeDtypeStruct((NACC, 8 * L), jnp.float32)],
        mesh=plsc.VectorSubcoreMesh(core_axis_name="c", subcore_axis_name="s"),
        compiler_params=pltpu.CompilerParams(use_tc_tiling_on_sc=False),
        scratch_types=[
            pltpu.VMEM_SHARED((NACC, 2 * L), jnp.float32),
            pltpu.VMEM((ZB_ROWS, 2 * L), jnp.float32),
            pltpu.VMEM((CH_Q, CHUNK), jnp.int32),
            pltpu.VMEM((CH_Q, CHUNK), jnp.int32),
            pltpu.VMEM((2 * GROWS, 2 * L), jnp.float32),
            pltpu.VMEM((WB_ROWS, 2 * L), jnp.float32),
            pltpu.SemaphoreType.DMA,
            pltpu.SemaphoreType.DMA,
        ],
    )
    return f(*gidx, dst2d, tab4)[0]


def _aggr_from(agg_ref, m_ref):
    eps = jnp.maximum(1e-16 * jnp.exp(-m_ref[0:1, :]), 1e-38)
    a = agg_ref[...]
    return jnp.concatenate(
        [a[:, cb * 2 * L + L:cb * 2 * L + 2 * L]
         / (a[:, cb * 2 * L:cb * 2 * L + L] + eps[:, cb * L:(cb + 1) * L])
         for cb in range(4)], axis=1)


def _layer_body(h_ref, m_ref, w1_ref, b1_ref, w2_ref, b2_ref, agg_ref,
                h_out, m_out, *, first):
    j = pl.program_id(0)
    aggr = _aggr_from(agg_ref, m_ref)
    h = h_ref[...]
    base = h if first else jax.nn.relu(h)
    u = base + aggr
    t = _dot(jax.nn.relu(_dot(u, w1_ref[...]) + b1_ref[0:1, :]),
             w2_ref[...]) + b2_ref[0:1, :]
    hnew = jax.nn.relu(t) if first else h + t
    h_out[...] = hnew
    bm = _colmax8(jax.nn.relu(hnew) + 1e-7)

    @pl.when(j == 0)
    def _():
        m_out[...] = bm

    @pl.when(j > 0)
    def _():
        m_out[...] = jnp.maximum(m_out[...], bm)


def _run_layer(h, M, W1, b1, W2, b2, agg, first):
    return pl.pallas_call(
        functools.partial(_layer_body, first=first),
        grid=(GRID,),
        in_specs=[
            pl.BlockSpec((BN, HID), lambda j: (j, 0)),
            pl.BlockSpec((8, HID), lambda j: (0, 0)),
            pl.BlockSpec((HID, FF), lambda j: (0, 0)),
            pl.BlockSpec((8, FF), lambda j: (0, 0)),
            pl.BlockSpec((FF, HID), lambda j: (0, 0)),
            pl.BlockSpec((8, HID), lambda j: (0, 0)),
            pl.BlockSpec((BN, 8 * L), lambda j: (j, 0)),
        ],
        out_specs=[
            pl.BlockSpec((BN, HID), lambda j: (j, 0)),
            pl.BlockSpec((8, HID), lambda j: (0, 0)),
        ],
        out_shape=[
            jax.ShapeDtypeStruct((N, HID), jnp.float32),
            jax.ShapeDtypeStruct((8, HID), jnp.float32),
        ],
    )(h, M, W1, b1, W2, b2, agg)


def _final_body(h_ref, m_ref, w1_ref, b1_ref, w2_ref, b2_ref,
                wh1_ref, bh1_ref, wh2_ref, bh2_ref, agg_ref,
                out_ref, pool_ref):
    j = pl.program_id(0)
    aggr = _aggr_from(agg_ref, m_ref)
    h = h_ref[...]
    u = jax.nn.relu(h) + aggr
    t = _dot(jax.nn.relu(_dot(u, w1_ref[...]) + b1_ref[0:1, :]),
             w2_ref[...]) + b2_ref[0:1, :]
    hnew = h + t
    bm = jnp.broadcast_to(jnp.max(hnew, axis=0, keepdims=True), (8, HID))

    @pl.when(j == 0)
    def _():
        pool_ref[...] = bm

    @pl.when(j > 0)
    def _():
        pool_ref[...] = jnp.maximum(pool_ref[...], bm)

    @pl.when(j == GRID - 1)
    def _():
        pooled = pool_ref[...]
        z = jax.nn.relu(_dot(pooled, wh1_ref[...]) + bh1_ref[0:1, :])
        out_ref[...] = _dot(z, wh2_ref[...]) + bh2_ref[0:1, :]


def _run_final(h, M, W1, b1, W2, b2, Wh1, bh1, Wh2, bh2, agg):
    return pl.pallas_call(
        _final_body,
        grid=(GRID,),
        in_specs=[
            pl.BlockSpec((BN, HID), lambda j: (j, 0)),
            pl.BlockSpec((8, HID), lambda j: (0, 0)),
            pl.BlockSpec((HID, FF), lambda j: (0, 0)),
            pl.BlockSpec((8, FF), lambda j: (0, 0)),
            pl.BlockSpec((FF, HID), lambda j: (0, 0)),
            pl.BlockSpec((8, HID), lambda j: (0, 0)),
            pl.BlockSpec((HID, HID), lambda j: (0, 0)),
            pl.BlockSpec((8, HID), lambda j: (0, 0)),
            pl.BlockSpec((HID, OUT_DIM), lambda j: (0, 0)),
            pl.BlockSpec((8, OUT_DIM), lambda j: (0, 0)),
            pl.BlockSpec((BN, 8 * L), lambda j: (j, 0)),
        ],
        out_specs=[pl.BlockSpec((8, OUT_DIM), lambda j: (0, 0))],
        out_shape=[jax.ShapeDtypeStruct((8, OUT_DIM), jnp.float32)],
        scratch_shapes=[pltpu.VMEM((8, HID), jnp.float32)],
    )(h, M, W1, b1, W2, b2, Wh1, bh1, Wh2, bh2, agg)


def _b8(b):
    return jnp.broadcast_to(b[None, :], (8, b.shape[0]))


def kernel(x, edge_index, Wl, bl, Win1, bin1, Win2, bin2,
           W0_1, b0_1, W0_2, b0_2, W1_1, b1_1, W1_2, b1_2,
           W2_1, b2_1, W2_2, b2_2, Wh1, bh1, Wh2, bh2):
    xp = jnp.pad(x, ((0, 0), (0, 2)))
    Wlp = jnp.pad(Wl, ((0, 2), (0, 0)))
    src = edge_index[0]
    dst = edge_index[1]
    pad = EPAD - src.shape[0]
    srcp = jnp.concatenate([src, jnp.zeros((pad,), jnp.int32)])
    gidx = tuple(
        (srcp * 4 + cb).reshape(CROWS, CHUNK) for cb in range(4))
    dst2d = jnp.concatenate(
        [dst, jnp.full((pad,), N, jnp.int32)]).reshape(CROWS, CHUNK)

    h, M = _run_k0(xp, Wlp, _b8(bl))

    layers = [
        (Win1, bin1, Win2, bin2),
        (W0_1, b0_1, W0_2, b0_2),
        (W1_1, b1_1, W1_2, b1_2),
        (W2_1, b2_1, W2_2, b2_2),
    ]
    for li, (W1, b1, W2, b2) in enumerate(layers):
        tab = _run_tbl(h, M)
        tab4 = tab.reshape(4 * N, 2 * L)
        agg = _run_sc(gidx, dst2d, tab4)
        if li < 3:
            h, M = _run_layer(h, M, W1, _b8(b1), W2, _b8(b2), agg,
                              first=(li == 0))
        else:
            out8 = _run_final(h, M, W1, _b8(b1), W2, _b8(b2),
                              Wh1, _b8(bh1), Wh2, _b8(bh2), agg)[0]
    return out8[0:1, :]


# speedup vs baseline: 14.0564x; 1.0179x over previous
import functools

import jax
import jax.numpy as jnp
from jax import lax
from jax.experimental import pallas as pl
from jax.experimental.pallas import tpu as pltpu
from jax.experimental.pallas import tpu_sc as plsc

N = 50000
HID = 64
FF = 128
OUT_DIM = 80

NC = 2
NS = 16
L = 16

BN = 2000
GRID = N // BN

CHUNK = 128
EPAD_UNIT = NS * CHUNK * 8
E_TOTAL = 800000
EPAD = ((E_TOTAL + EPAD_UNIT - 1) // EPAD_UNIT) * EPAD_UNIT
CROWS = EPAD // CHUNK
CROWS_TILE = CROWS // NS
STAGES = 28
CH_Q = CROWS_TILE // STAGES
GRP = 2
NGRP = CH_Q // GRP
GROWS = GRP * CHUNK
NSLOT = 3

ACC_TILE = 3136
NACC = NS * ACC_TILE
ZB_ROWS = 448
ZB_STEPS = ACC_TILE // ZB_ROWS
WB_ROWS = 224
WB_STEPS = ACC_TILE // WB_ROWS


def _dot(a, b):
    return jnp.dot(a, b, preferred_element_type=jnp.float32)


def _colmax8(v):
    return jnp.broadcast_to(jnp.max(v, axis=0, keepdims=True), (8, HID))


def _k0_body(x_ref, w_ref, b_ref, h_ref, m_ref):
    j = pl.program_id(0)
    h = _dot(x_ref[...], w_ref[...]) + b_ref[0:1, :]
    h_ref[...] = h
    bm = _colmax8(jax.nn.relu(h) + 1e-7)

    @pl.when(j == 0)
    def _():
        m_ref[...] = bm

    @pl.when(j > 0)
    def _():
        m_ref[...] = jnp.maximum(m_ref[...], bm)


def _run_k0(xp, Wlp, bl2):
    return pl.pallas_call(
        _k0_body,
        grid=(GRID,),
        in_specs=[
            pl.BlockSpec((BN, 8), lambda j: (j, 0)),
            pl.BlockSpec((8, HID), lambda j: (0, 0)),
            pl.BlockSpec((8, HID), lambda j: (0, 0)),
        ],
        out_specs=[
            pl.BlockSpec((BN, HID), lambda j: (j, 0)),
            pl.BlockSpec((8, HID), lambda j: (0, 0)),
        ],
        out_shape=[
            jax.ShapeDtypeStruct((N, HID), jnp.float32),
            jax.ShapeDtypeStruct((8, HID), jnp.float32),
        ],
    )(xp, Wlp, bl2)


def _tbl_body(h_ref, m_ref, t_ref):
    v = jax.nn.relu(h_ref[...]) + 1e-7
    w = jnp.exp(v - m_ref[0:1, :])
    wv = w * v
    parts = []
    for cb in range(4):
        parts.append(w[:, cb * L:(cb + 1) * L])
        parts.append(wv[:, cb * L:(cb + 1) * L])
    t_ref[...] = jnp.concatenate(parts, axis=1)


def _run_tbl(h, M):
    return pl.pallas_call(
        _tbl_body,
        grid=(GRID,),
        in_specs=[
            pl.BlockSpec((BN, HID), lambda j: (j, 0)),
            pl.BlockSpec((8, HID), lambda j: (0, 0)),
        ],
        out_specs=[pl.BlockSpec((BN, 2 * HID), lambda j: (j, 0))],
        out_shape=[jax.ShapeDtypeStruct((N, 2 * HID), jnp.float32)],
    )(h, M)[0]


def _sc_body(src_ref, dst_ref, tab_ref, agg_ref,
             accC, sidx, didx, gb, semg, sems):
    c = lax.axis_index("c")
    s = lax.axis_index("s")
    row0 = s * ACC_TILE
    chunk0 = s * CROWS_TILE

    def _fire_gathers(g, slot):
        off = slot * GROWS
        for j in range(GRP):
            r = g * GRP + j
            pltpu.async_copy(tab_ref.at[sidx.at[r]],
                             gb.at[pl.ds(off + j * CHUNK, CHUNK)], semg)

    def _fire_scatters(g, slot):
        off = slot * GROWS
        for j in range(GRP):
            r = g * GRP + j
            pltpu.async_copy(gb.at[pl.ds(off + j * CHUNK, CHUNK)],
                             accC.at[didx.at[r]], sems, add=True)

    def _drain(sem, rows):
        pltpu.make_async_copy(tab_ref.at[pl.ds(0, rows)],
                              gb.at[pl.ds(0, rows)], sem).wait()

    def _edges(cb):
        def stage(q, _):
            r0 = chunk0 + q * CH_Q
            pltpu.sync_copy(src_ref.at[pl.ds(r0, CH_Q)], sidx)
            pltpu.sync_copy(dst_ref.at[pl.ds(r0, CH_Q)], didx)
            for i in range(CH_Q):
                for j2 in range(CHUNK // L):
                    sl = sidx[i, pl.ds(j2 * L, L)]
                    sidx[i, pl.ds(j2 * L, L)] = sl * 4 + cb
            _fire_gathers(0, 0)
            _fire_gathers(1, 1)

            def grp(g, _):
                slot = lax.rem(g, NSLOT)
                _drain(semg, GROWS)
                _fire_scatters(g, slot)

                @pl.when(g > 0)
                def _():
                    _drain(sems, GROWS)

                @pl.when(g < NGRP - 2)
                def _():
                    _fire_gathers(g + 2, lax.rem(g + 2, NSLOT))
                return _
            lax.fori_loop(0, NGRP, grp, None)
            _drain(sems, GROWS)
            return _
        lax.fori_loop(0, STAGES, stage, None)

    def _zero():
        def zf(i, _):
            gb[i, 0:L] = jnp.zeros((L,), jnp.float32)
            gb[i, L:2 * L] = jnp.zeros((L,), jnp.float32)
            return _
        lax.fori_loop(0, ZB_ROWS, zf, None)
        for k in range(ZB_STEPS):
            pltpu.async_copy(gb.at[pl.ds(0, ZB_ROWS)],
                             accC.at[pl.ds(row0 + k * ZB_ROWS, ZB_ROWS)], semg)
        _drain(semg, ACC_TILE)

    def _writeback(cb):
        pltpu.async_copy(accC.at[pl.ds(row0, WB_ROWS)],
                         gb.at[pl.ds(0, WB_ROWS)], semg)

        def wr(k, _):
            off = lax.rem(k, 2) * GROWS

            @pl.when(k > 0)
            def _():
                _drain(sems, WB_ROWS)

            @pl.when(k < WB_STEPS - 1)
            def _():
                pltpu.async_copy(
                    accC.at[pl.ds(row0 + (k + 1) * WB_ROWS, WB_ROWS)],
                    gb.at[pl.ds((lax.rem(k, 2) ^ 1) * GROWS, WB_ROWS)], semg)
            _drain(semg, WB_ROWS)
            pltpu.async_copy(
                gb.at[pl.ds(off, WB_ROWS)],
                agg_ref.at[pl.ds(row0 + k * WB_ROWS, WB_ROWS),
                           pl.ds(cb * 2 * L, 2 * L)], sems)
            return _
        lax.fori_loop(0, WB_STEPS, wr, None)
        _drain(sems, WB_ROWS)

    for phase in range(2):
        _zero()
        plsc.subcore_barrier()

        for cc in range(NC):
            cb = 2 * cc + phase

            @pl.when(c == cc)
            def _(cb=cb):
                _edges(cb)
        plsc.subcore_barrier()

        for cc in range(NC):
            cb = 2 * cc + phase

            @pl.when(c == cc)
            def _(cb=cb):
                _writeback(cb)
        plsc.subcore_barrier()


def _run_sc(src2d, dst2d, tab4):
    f = pl.kernel(
        _sc_body,
        out_type=[jax.ShapeDtypeStruct((NACC, 8 * L), jnp.float32)],
        mesh=plsc.VectorSubcoreMesh(core_axis_name="c", subcore_axis_name="s"),
        compiler_params=pltpu.CompilerParams(use_tc_tiling_on_sc=False),
        scratch_types=[
            pltpu.VMEM_SHARED((NACC, 2 * L), jnp.float32),
            pltpu.VMEM((CH_Q, CHUNK), jnp.int32),
            pltpu.VMEM((CH_Q, CHUNK), jnp.int32),
            pltpu.VMEM((NSLOT * GROWS, 2 * L), jnp.float32),
            pltpu.SemaphoreType.DMA,
            pltpu.SemaphoreType.DMA,
        ],
    )
    return f(src2d, dst2d, tab4)[0]


def _aggr_from(agg_ref, m_ref):
    eps = jnp.maximum(1e-16 * jnp.exp(-m_ref[0:1, :]), 1e-38)
    a = agg_ref[...]
    return jnp.concatenate(
        [a[:, cb * 2 * L + L:cb * 2 * L + 2 * L]
         / (a[:, cb * 2 * L:cb * 2 * L + L] + eps[:, cb * L:(cb + 1) * L])
         for cb in range(4)], axis=1)


def _layer_body(h_ref, m_ref, w1_ref, b1_ref, w2_ref, b2_ref, agg_ref,
                h_out, m_out, *, first):
    j = pl.program_id(0)
    aggr = _aggr_from(agg_ref, m_ref)
    h = h_ref[...]
    base = h if first else jax.nn.relu(h)
    u = base + aggr
    t = _dot(jax.nn.relu(_dot(u, w1_ref[...]) + b1_ref[0:1, :]),
             w2_ref[...]) + b2_ref[0:1, :]
    hnew = jax.nn.relu(t) if first else h + t
    h_out[...] = hnew
    bm = _colmax8(jax.nn.relu(hnew) + 1e-7)

    @pl.when(j == 0)
    def _():
        m_out[...] = bm

    @pl.when(j > 0)
    def _():
        m_out[...] = jnp.maximum(m_out[...], bm)


def _run_layer(h, M, W1, b1, W2, b2, agg, first):
    return pl.pallas_call(
        functools.partial(_layer_body, first=first),
        grid=(GRID,),
        in_specs=[
            pl.BlockSpec((BN, HID), lambda j: (j, 0)),
            pl.BlockSpec((8, HID), lambda j: (0, 0)),
            pl.BlockSpec((HID, FF), lambda j: (0, 0)),
            pl.BlockSpec((8, FF), lambda j: (0, 0)),
            pl.BlockSpec((FF, HID), lambda j: (0, 0)),
            pl.BlockSpec((8, HID), lambda j: (0, 0)),
            pl.BlockSpec((BN, 8 * L), lambda j: (j, 0)),
        ],
        out_specs=[
            pl.BlockSpec((BN, HID), lambda j: (j, 0)),
            pl.BlockSpec((8, HID), lambda j: (0, 0)),
        ],
        out_shape=[
            jax.ShapeDtypeStruct((N, HID), jnp.float32),
            jax.ShapeDtypeStruct((8, HID), jnp.float32),
        ],
    )(h, M, W1, b1, W2, b2, agg)


def _final_body(h_ref, m_ref, w1_ref, b1_ref, w2_ref, b2_ref,
                wh1_ref, bh1_ref, wh2_ref, bh2_ref, agg_ref,
                out_ref, pool_ref):
    j = pl.program_id(0)
    aggr = _aggr_from(agg_ref, m_ref)
    h = h_ref[...]
    u = jax.nn.relu(h) + aggr
    t = _dot(jax.nn.relu(_dot(u, w1_ref[...]) + b1_ref[0:1, :]),
             w2_ref[...]) + b2_ref[0:1, :]
    hnew = h + t
    bm = jnp.broadcast_to(jnp.max(hnew, axis=0, keepdims=True), (8, HID))

    @pl.when(j == 0)
    def _():
        pool_ref[...] = bm

    @pl.when(j > 0)
    def _():
        pool_ref[...] = jnp.maximum(pool_ref[...], bm)

    @pl.when(j == GRID - 1)
    def _():
        pooled = pool_ref[...]
        z = jax.nn.relu(_dot(pooled, wh1_ref[...]) + bh1_ref[0:1, :])
        out_ref[...] = _dot(z, wh2_ref[...]) + bh2_ref[0:1, :]


def _run_final(h, M, W1, b1, W2, b2, Wh1, bh1, Wh2, bh2, agg):
    return pl.pallas_call(
        _final_body,
        grid=(GRID,),
        in_specs=[
            pl.BlockSpec((BN, HID), lambda j: (j, 0)),
            pl.BlockSpec((8, HID), lambda j: (0, 0)),
            pl.BlockSpec((HID, FF), lambda j: (0, 0)),
            pl.BlockSpec((8, FF), lambda j: (0, 0)),
            pl.BlockSpec((FF, HID), lambda j: (0, 0)),
            pl.BlockSpec((8, HID), lambda j: (0, 0)),
            pl.BlockSpec((HID, HID), lambda j: (0, 0)),
            pl.BlockSpec((8, HID), lambda j: (0, 0)),
            pl.BlockSpec((HID, OUT_DIM), lambda j: (0, 0)),
            pl.BlockSpec((8, OUT_DIM), lambda j: (0, 0)),
            pl.BlockSpec((BN, 8 * L), lambda j: (j, 0)),
        ],
        out_specs=[pl.BlockSpec((8, OUT_DIM), lambda j: (0, 0))],
        out_shape=[jax.ShapeDtypeStruct((8, OUT_DIM), jnp.float32)],
        scratch_shapes=[pltpu.VMEM((8, HID), jnp.float32)],
    )(h, M, W1, b1, W2, b2, Wh1, bh1, Wh2, bh2, agg)


def _b8(b):
    return jnp.broadcast_to(b[None, :], (8, b.shape[0]))


def kernel(x, edge_index, Wl, bl, Win1, bin1, Win2, bin2,
           W0_1, b0_1, W0_2, b0_2, W1_1, b1_1, W1_2, b1_2,
           W2_1, b2_1, W2_2, b2_2, Wh1, bh1, Wh2, bh2):
    xp = jnp.pad(x, ((0, 0), (0, 2)))
    Wlp = jnp.pad(Wl, ((0, 2), (0, 0)))
    src = edge_index[0]
    dst = edge_index[1]
    pad = EPAD - src.shape[0]
    src2d = jnp.concatenate(
        [src, jnp.zeros((pad,), jnp.int32)]).reshape(CROWS, CHUNK)
    dst2d = jnp.concatenate(
        [dst, jnp.full((pad,), N, jnp.int32)]).reshape(CROWS, CHUNK)

    h, M = _run_k0(xp, Wlp, _b8(bl))

    layers = [
        (Win1, bin1, Win2, bin2),
        (W0_1, b0_1, W0_2, b0_2),
        (W1_1, b1_1, W1_2, b1_2),
        (W2_1, b2_1, W2_2, b2_2),
    ]
    for li, (W1, b1, W2, b2) in enumerate(layers):
        tab = _run_tbl(h, M)
        tab4 = tab.reshape(4 * N, 2 * L)
        agg = _run_sc(src2d, dst2d, tab4)
        if li < 3:
            h, M = _run_layer(h, M, W1, _b8(b1), W2, _b8(b2), agg,
                              first=(li == 0))
        else:
            out8 = _run_final(h, M, W1, _b8(b1), W2, _b8(b2),
                              Wh1, _b8(bh1), Wh2, _b8(bh2), agg)[0]
    return out8[0:1, :]
